# baseline (speedup 1.0000x reference)
import functools

import jax
import jax.numpy as jnp
from jax import lax
from jax.experimental import pallas as pl
from jax.experimental.pallas import tpu as pltpu
from jax.experimental.pallas import tpu_sc as plsc

_N = 10000
_E = 320000
_D = 128
_H = 64
_C = 10
_G = 64

_NC = 2
_NS = 16
_NW = _NC * _NS
_CH = 128
_CPW = 80
_NWC = _NW * _CPW
_EPAD = _NWC * _CH
_NPAD = 10112
_RPT = _NPAD // _NS
_DEGW = 16

_R = 1000
_NB = _N // _R

@functools.cache
def _mesh():
    return plsc.VectorSubcoreMesh(core_axis_name="c", subcore_axis_name="s",
                                  num_cores=_NC, num_subcores=_NS)


def _zero_shared_slice(zb, sh, row0):
    for k in range(4):
        pltpu.sync_copy(zb, sh.at[pl.ds(row0 + k * 128, 128)])
    pltpu.sync_copy(zb.at[pl.ds(0, 120)], sh.at[pl.ds(row0 + 512, 120)])


def _fill_const(ref, rows, width, value):
    def body(r, _):
        for k in range(width // 16):
            ref[r, pl.ds(k * 16, 16)] = jnp.full((16,), value, jnp.float32)
        return 0
    lax.fori_loop(0, rows, body, 0)


_DG = 8
_NGD = _CPW // _DG


def _sc_deg_body(e_hbm, out_hbm, deg_sh, ones_v, zb, idx_a, idx_b, ssem):
    c = lax.axis_index("c")
    s = lax.axis_index("s")
    wid = c * _NS + s
    _fill_const(ones_v, _CH, _DEGW, 1.0)
    _fill_const(zb, _CH, _DEGW, 0.0)
    _zero_shared_slice(zb, deg_sh, s * _RPT)
    plsc.subcore_barrier()
    base = wid * _CPW
    idx = (idx_a, idx_b)

    pltpu.sync_copy(e_hbm.at[pl.ds(base, _DG)], idx_a)

    def step2(k, _):
        for b in (0, 1):
            g = 2 * k + b
            p, pn = b, 1 - b

            @pl.when(g > 0)
            def _():
                for q in range(_DG):
                    pltpu.make_async_copy(
                        ones_v, deg_sh.at[idx[pn].at[q, 1]], ssem).wait()

            pltpu.sync_copy(e_hbm.at[pl.ds(base + (g + 1) * _DG, _DG)],
                            idx[pn])
            for q in range(_DG):
                pltpu.async_copy(ones_v, deg_sh.at[idx[p].at[q, 1]], ssem,
                                 add=True)
        return 0

    lax.fori_loop(0, _NGD // 2, step2, 0)
    for q in range(_DG):
        pltpu.make_async_copy(ones_v, deg_sh.at[idx_b.at[q, 1]], ssem).wait()
    plsc.subcore_barrier()
    pltpu.sync_copy(deg_sh.at[pl.ds(s * _RPT, _RPT)],
                    out_hbm.at[c, pl.ds(s * _RPT, _RPT)])


@functools.cache
def _sc_deg():
    return pl.kernel(
        _sc_deg_body,
        out_type=jax.ShapeDtypeStruct((_NC, _NPAD, _DEGW), jnp.float32),
        mesh=_mesh(),
        compiler_params=pltpu.CompilerParams(use_tc_tiling_on_sc=False),
        scratch_types=[
            pltpu.VMEM_SHARED((_NPAD, _DEGW), jnp.float32),
            pltpu.VMEM((_CH, _DEGW), jnp.float32),
            pltpu.VMEM((_CH, _DEGW), jnp.float32),
            pltpu.VMEM((_DG, 2, _CH), jnp.int32),
            pltpu.VMEM((_DG, 2, _CH), jnp.int32),
            pltpu.SemaphoreType.DMA,
        ],
    )


_GRP = 4
_NG = _CPW // _GRP


def _sc_agg_body(u_hbm, e_hbm, out_hbm, agg_sh, zb, idx_a, idx_b, rows_a,
                 rows_b, gsem_a, gsem_b, ssem_a, ssem_b):
    c = lax.axis_index("c")
    s = lax.axis_index("s")
    wid = c * _NS + s
    _fill_const(zb, _CH, _H, 0.0)
    _zero_shared_slice(zb, agg_sh, s * _RPT)
    plsc.subcore_barrier()
    base = wid * _CPW
    idx = (idx_a, idx_b)
    rows = (rows_a, rows_b)
    gsem = (gsem_a, gsem_b)
    ssem = (ssem_a, ssem_b)

    def fire_gathers(p):
        for q in range(_GRP):
            pltpu.async_copy(u_hbm.at[idx[p].at[q, 0]],
                             rows[p].at[pl.ds(q * _CH, _CH)], gsem[p])

    def wait_gathers(p):
        for q in range(_GRP):
            pltpu.make_async_copy(u_hbm.at[idx[p].at[q, 0]],
                                  rows[p].at[pl.ds(q * _CH, _CH)],
                                  gsem[p]).wait()

    def fire_scatters(p):
        for q in range(_GRP):
            pltpu.async_copy(rows[p].at[pl.ds(q * _CH, _CH)],
                             agg_sh.at[idx[p].at[q, 1]], ssem[p], add=True)

    def wait_scatters(p):
        for q in range(_GRP):
            pltpu.make_async_copy(rows[p].at[pl.ds(q * _CH, _CH)],
                                  agg_sh.at[idx[p].at[q, 1]], ssem[p]).wait()

    pltpu.sync_copy(e_hbm.at[pl.ds(base, _GRP)], idx_a)
    fire_gathers(0)

    def step2(k, _):
        for b in (0, 1):
            g = 2 * k + b
            p, pn = b, 1 - b

            @pl.when(g > 0)
            def _():
                wait_scatters(pn)

            pltpu.sync_copy(e_hbm.at[pl.ds(base + (g + 1) * _GRP, _GRP)],
                            idx[pn])
            fire_gathers(pn)
            wait_gathers(p)
            fire_scatters(p)
        return 0

    lax.fori_loop(0, _NG // 2, step2, 0)
    wait_scatters(1)
    wait_gathers(0)
    plsc.subcore_barrier()
    pltpu.sync_copy(agg_sh.at[pl.ds(s * _RPT, _RPT)],
                    out_hbm.at[c, pl.ds(s * _RPT, _RPT)])


@functools.cache
def _sc_agg():
    return pl.kernel(
        _sc_agg_body,
        out_type=jax.ShapeDtypeStruct((_NC, _NPAD, _H), jnp.float32),
        mesh=_mesh(),
        compiler_params=pltpu.CompilerParams(use_tc_tiling_on_sc=False),
        scratch_types=[
            pltpu.VMEM_SHARED((_NPAD, _H), jnp.float32),
            pltpu.VMEM((_CH, _H), jnp.float32),
            pltpu.VMEM((_GRP, 2, _CH), jnp.int32),
            pltpu.VMEM((_GRP, 2, _CH), jnp.int32),
            pltpu.VMEM((_GRP * _CH, _H), jnp.float32),
            pltpu.VMEM((_GRP * _CH, _H), jnp.float32),
            pltpu.SemaphoreType.DMA,
            pltpu.SemaphoreType.DMA,
            pltpu.SemaphoreType.DMA,
            pltpu.SemaphoreType.DMA,
        ],
    )


def _tc1_body(x_ref, degp_ref, w1_ref, u1_ref, dinv_ref):
    deg = degp_ref[0, :, 0:1] + degp_ref[1, :, 0:1] + 1.0
    dinv = lax.rsqrt(deg)
    h = jnp.dot(x_ref[...], w1_ref[...], preferred_element_type=jnp.float32)
    u1_ref[...] = h * dinv
    dinv_ref[...] = dinv


_tc1 = pl.pallas_call(
    _tc1_body,
    grid=(_NB,),
    in_specs=[
        pl.BlockSpec((_R, _D), lambda i: (i, 0)),
        pl.BlockSpec((_NC, _R, _DEGW), lambda i: (0, i, 0)),
        pl.BlockSpec((_D, _H), lambda i: (0, 0)),
    ],
    out_specs=[
        pl.BlockSpec((_R, _H), lambda i: (i, 0)),
        pl.BlockSpec((_R, 1), lambda i: (i, 0)),
    ],
    out_shape=[
        jax.ShapeDtypeStruct((_N, _H), jnp.float32),
        jax.ShapeDtypeStruct((_N, 1), jnp.float32),
    ],
)


def _tc2_body(aggp_ref, u1_ref, dinv_ref, b1_ref, w2_ref, u2_ref):
    t = (aggp_ref[0] + aggp_ref[1] + u1_ref[...]) * dinv_ref[...] + b1_ref[...]
    t = jnp.maximum(t, 0.0)
    h2 = jnp.dot(t, w2_ref[...], preferred_element_type=jnp.float32)
    u2_ref[...] = h2 * dinv_ref[...]


_tc2 = pl.pallas_call(
    _tc2_body,
    grid=(_NB,),
    in_specs=[
        pl.BlockSpec((_NC, _R, _H), lambda i: (0, i, 0)),
        pl.BlockSpec((_R, _H), lambda i: (i, 0)),
        pl.BlockSpec((_R, 1), lambda i: (i, 0)),
        pl.BlockSpec((1, _H), lambda i: (0, 0)),
        pl.BlockSpec((_H, _H), lambda i: (0, 0)),
    ],
    out_specs=pl.BlockSpec((_R, _H), lambda i: (i, 0)),
    out_shape=jax.ShapeDtypeStruct((_N, _H), jnp.float32),
)


def _tc3_body(aggp_ref, u2_ref, dinv_ref, b2_ref, batch_ref, sums_ref, cnts_ref):
    i = pl.program_id(0)
    out2 = (aggp_ref[0] + aggp_ref[1] + u2_ref[...]) * dinv_ref[...] + b2_ref[...]
    out2 = jnp.maximum(out2, 0.0)
    b = batch_ref[0, 0, :]
    gids = lax.broadcasted_iota(jnp.int32, (_G, _R), 0)
    onehot = (b[None, :] == gids).astype(jnp.float32)
    psums = jnp.dot(onehot, out2, preferred_element_type=jnp.float32)
    pcnts = jnp.sum(onehot, axis=1, keepdims=True)

    @pl.when(i == 0)
    def _():
        sums_ref[...] = psums
        cnts_ref[...] = pcnts

    @pl.when(i > 0)
    def _():
        sums_ref[...] += psums
        cnts_ref[...] += pcnts


_tc3 = pl.pallas_call(
    _tc3_body,
    grid=(_NB,),
    in_specs=[
        pl.BlockSpec((_NC, _R, _H), lambda i: (0, i, 0)),
        pl.BlockSpec((_R, _H), lambda i: (i, 0)),
        pl.BlockSpec((_R, 1), lambda i: (i, 0)),
        pl.BlockSpec((1, _H), lambda i: (0, 0)),
        pl.BlockSpec((1, 1, _R), lambda i: (i, 0, 0)),
    ],
    out_specs=[
        pl.BlockSpec((_G, _H), lambda i: (0, 0)),
        pl.BlockSpec((_G, 1), lambda i: (0, 0)),
    ],
    out_shape=[
        jax.ShapeDtypeStruct((_G, _H), jnp.float32),
        jax.ShapeDtypeStruct((_G, 1), jnp.float32),
    ],
)


def _tc4_body(sums_ref, cnts_ref, l1w_ref, l1b_ref, l2w_ref, l2b_ref,
              l3w_ref, l3b_ref, out_ref):
    g = sums_ref[...] / jnp.maximum(cnts_ref[...], 1.0)
    g = jnp.maximum(
        jnp.dot(g, l1w_ref[...], preferred_element_type=jnp.float32)
        + l1b_ref[...], 0.0)
    g = jnp.maximum(
        jnp.dot(g, l2w_ref[...], preferred_element_type=jnp.float32)
        + l2b_ref[...], 0.0)
    logits = (jnp.dot(g, l3w_ref[...], preferred_element_type=jnp.float32)
              + l3b_ref[...])
    m = jnp.max(logits, axis=-1, keepdims=True)
    lse = m + jnp.log(jnp.sum(jnp.exp(logits - m), axis=-1, keepdims=True))
    out_ref[...] = logits - lse


_tc4 = pl.pallas_call(
    _tc4_body,
    out_shape=jax.ShapeDtypeStruct((_G, _C), jnp.float32),
)


def kernel(x, edge_index, batch, W1, b1, W2, b2, lin1_W, lin1_b, lin2_W,
           lin2_b, lin3_W, lin3_b):
    src = edge_index[0]
    dst = edge_index[1]
    pad = _EPAD - _E
    srcp = jnp.concatenate([src, jnp.zeros((pad,), jnp.int32)]).reshape(_NWC, _CH)
    dstp = jnp.concatenate([dst, jnp.full((pad,), _N, jnp.int32)]).reshape(_NWC, _CH)
    slack = jnp.zeros((_DG, 2, _CH), jnp.int32).at[:, 1, :].set(_N)
    earr = jnp.concatenate([jnp.stack([srcp, dstp], axis=1), slack], axis=0)

    degp = _sc_deg()(earr)
    u1, dinv = _tc1(x, degp, W1)
    agg1 = _sc_agg()(u1, earr)
    u2 = _tc2(agg1, u1, dinv, b1.reshape(1, _H), W2)
    agg2 = _sc_agg()(u2, earr)
    sums, cnts = _tc3(agg2, u2, dinv, b2.reshape(1, _H),
                      batch.reshape(_NB, 1, _R))
    out = _tc4(sums, cnts, lin1_W, lin1_b.reshape(1, _H), lin2_W,
               lin2_b.reshape(1, _H // 2), lin3_W, lin3_b.reshape(1, _C))
    return out


# speedup vs baseline: 1.0063x; 1.0063x over previous
import functools

import jax
import jax.numpy as jnp
from jax import lax
from jax.experimental import pallas as pl
from jax.experimental.pallas import tpu as pltpu
from jax.experimental.pallas import tpu_sc as plsc

_N = 10000
_E = 320000
_D = 128
_H = 64
_C = 10
_G = 64

_NC = 2
_NS = 16
_NW = _NC * _NS
_CH = 128
_CPW = 80
_NWC = _NW * _CPW
_EPAD = _NWC * _CH
_NPAD = 10112
_RPT = _NPAD // _NS
_DEGW = 16

_R = 1000
_NB = _N // _R

@functools.cache
def _mesh():
    return plsc.VectorSubcoreMesh(core_axis_name="c", subcore_axis_name="s",
                                  num_cores=_NC, num_subcores=_NS)


def _zero_shared_slice(zb, sh, row0):
    for k in range(4):
        pltpu.sync_copy(zb, sh.at[pl.ds(row0 + k * 128, 128)])
    pltpu.sync_copy(zb.at[pl.ds(0, 120)], sh.at[pl.ds(row0 + 512, 120)])


def _fill_const(ref, rows, width, value):
    def body(r, _):
        for k in range(width // 16):
            ref[r, pl.ds(k * 16, 16)] = jnp.full((16,), value, jnp.float32)
        return 0
    lax.fori_loop(0, rows, body, 0)


_DG = 8
_NGD = _CPW // _DG


def _sc_deg_body(e_hbm, out_hbm, deg_sh, ones_v, zb, idx_a, idx_b, ssem):
    c = lax.axis_index("c")
    s = lax.axis_index("s")
    wid = c * _NS + s
    _fill_const(ones_v, _CH, _DEGW, 1.0)
    _fill_const(zb, _CH, _DEGW, 0.0)
    _zero_shared_slice(zb, deg_sh, s * _RPT)
    plsc.subcore_barrier()
    base = wid * _CPW
    idx = (idx_a, idx_b)

    pltpu.sync_copy(e_hbm.at[pl.ds(base, _DG)], idx_a)

    def step2(k, _):
        for b in (0, 1):
            g = 2 * k + b
            p, pn = b, 1 - b

            @pl.when(g > 0)
            def _():
                for q in range(_DG):
                    pltpu.make_async_copy(
                        ones_v, deg_sh.at[idx[pn].at[q, 1]], ssem).wait()

            pltpu.sync_copy(e_hbm.at[pl.ds(base + (g + 1) * _DG, _DG)],
                            idx[pn])
            for q in range(_DG):
                pltpu.async_copy(ones_v, deg_sh.at[idx[p].at[q, 1]], ssem,
                                 add=True)
        return 0

    lax.fori_loop(0, _NGD // 2, step2, 0)
    for q in range(_DG):
        pltpu.make_async_copy(ones_v, deg_sh.at[idx_b.at[q, 1]], ssem).wait()
    plsc.subcore_barrier()
    pltpu.sync_copy(deg_sh.at[pl.ds(s * _RPT, _RPT)],
                    out_hbm.at[c, pl.ds(s * _RPT, _RPT)])


@functools.cache
def _sc_deg():
    return pl.kernel(
        _sc_deg_body,
        out_type=jax.ShapeDtypeStruct((_NC, _NPAD, _DEGW), jnp.float32),
        mesh=_mesh(),
        compiler_params=pltpu.CompilerParams(use_tc_tiling_on_sc=False),
        scratch_types=[
            pltpu.VMEM_SHARED((_NPAD, _DEGW), jnp.float32),
            pltpu.VMEM((_CH, _DEGW), jnp.float32),
            pltpu.VMEM((_CH, _DEGW), jnp.float32),
            pltpu.VMEM((_DG, 2, _CH), jnp.int32),
            pltpu.VMEM((_DG, 2, _CH), jnp.int32),
            pltpu.SemaphoreType.DMA,
        ],
    )


_GRP = 4
_NG = _CPW // _GRP


def _sc_agg_body(u_hbm, e_hbm, out_hbm, agg_sh, zb, idx_a, idx_b, rows_a,
                 rows_b, gsem_a, gsem_b, ssem_a, ssem_b):
    c = lax.axis_index("c")
    s = lax.axis_index("s")
    wid = c * _NS + s
    _fill_const(zb, _CH, _H, 0.0)
    _zero_shared_slice(zb, agg_sh, s * _RPT)
    plsc.subcore_barrier()
    base = wid * _CPW
    idx = (idx_a, idx_b)
    rows = (rows_a, rows_b)
    gsem = (gsem_a, gsem_b)
    ssem = (ssem_a, ssem_b)

    def fire_gathers(p):
        for q in range(_GRP):
            pltpu.async_copy(u_hbm.at[idx[p].at[q, 0]],
                             rows[p].at[pl.ds(q * _CH, _CH)], gsem[p])

    def wait_gathers(p):
        for q in range(_GRP):
            pltpu.make_async_copy(u_hbm.at[idx[p].at[q, 0]],
                                  rows[p].at[pl.ds(q * _CH, _CH)],
                                  gsem[p]).wait()

    def fire_scatters(p):
        for q in range(_GRP):
            pltpu.async_copy(rows[p].at[pl.ds(q * _CH, _CH)],
                             agg_sh.at[idx[p].at[q, 1]], ssem[p], add=True)

    def wait_scatters(p):
        for q in range(_GRP):
            pltpu.make_async_copy(rows[p].at[pl.ds(q * _CH, _CH)],
                                  agg_sh.at[idx[p].at[q, 1]], ssem[p]).wait()

    pltpu.sync_copy(e_hbm.at[pl.ds(base, _GRP)], idx_a)
    fire_gathers(0)

    def step2(k, _):
        for b in (0, 1):
            g = 2 * k + b
            p, pn = b, 1 - b

            @pl.when(g > 0)
            def _():
                wait_scatters(pn)

            pltpu.sync_copy(e_hbm.at[pl.ds(base + (g + 1) * _GRP, _GRP)],
                            idx[pn])
            fire_gathers(pn)
            wait_gathers(p)
            fire_scatters(p)
        return 0

    lax.fori_loop(0, _NG // 2, step2, 0)
    wait_scatters(1)
    wait_gathers(0)
    plsc.subcore_barrier()
    pltpu.sync_copy(agg_sh.at[pl.ds(s * _RPT, _RPT)],
                    out_hbm.at[c, pl.ds(s * _RPT, _RPT)])


@functools.cache
def _sc_agg():
    return pl.kernel(
        _sc_agg_body,
        out_type=jax.ShapeDtypeStruct((_NC, _NPAD, _H), jnp.float32),
        mesh=_mesh(),
        compiler_params=pltpu.CompilerParams(use_tc_tiling_on_sc=False),
        scratch_types=[
            pltpu.VMEM_SHARED((_NPAD, _H), jnp.float32),
            pltpu.VMEM((_CH, _H), jnp.float32),
            pltpu.VMEM((_GRP, 2, _CH), jnp.int32),
            pltpu.VMEM((_GRP, 2, _CH), jnp.int32),
            pltpu.VMEM((_GRP * _CH, _H), jnp.float32),
            pltpu.VMEM((_GRP * _CH, _H), jnp.float32),
            pltpu.SemaphoreType.DMA,
            pltpu.SemaphoreType.DMA,
            pltpu.SemaphoreType.DMA,
            pltpu.SemaphoreType.DMA,
        ],
    )


def _tc1_body(x_ref, degp_ref, w1_ref, u1_ref, dinv_ref):
    deg = degp_ref[0, :, 0:1] + degp_ref[1, :, 0:1] + 1.0
    dinv = lax.rsqrt(deg)
    h = jnp.dot(x_ref[...], w1_ref[...], preferred_element_type=jnp.float32)
    u1_ref[...] = h * dinv
    dinv_ref[...] = dinv


_tc1 = pl.pallas_call(
    _tc1_body,
    grid=(_NB,),
    in_specs=[
        pl.BlockSpec((_R, _D), lambda i: (i, 0)),
        pl.BlockSpec((_NC, _R, _DEGW), lambda i: (0, i, 0)),
        pl.BlockSpec((_D, _H), lambda i: (0, 0)),
    ],
    out_specs=[
        pl.BlockSpec((_R, _H), lambda i: (i, 0)),
        pl.BlockSpec((_R, 1), lambda i: (i, 0)),
    ],
    out_shape=[
        jax.ShapeDtypeStruct((_N, _H), jnp.float32),
        jax.ShapeDtypeStruct((_N, 1), jnp.float32),
    ],
)


def _tc2_body(aggp_ref, u1_ref, dinv_ref, b1_ref, w2_ref, u2_ref):
    t = (aggp_ref[0] + aggp_ref[1] + u1_ref[...]) * dinv_ref[...] + b1_ref[...]
    t = jnp.maximum(t, 0.0)
    h2 = jnp.dot(t, w2_ref[...], preferred_element_type=jnp.float32)
    u2_ref[...] = h2 * dinv_ref[...]


_tc2 = pl.pallas_call(
    _tc2_body,
    grid=(_NB,),
    in_specs=[
        pl.BlockSpec((_NC, _R, _H), lambda i: (0, i, 0)),
        pl.BlockSpec((_R, _H), lambda i: (i, 0)),
        pl.BlockSpec((_R, 1), lambda i: (i, 0)),
        pl.BlockSpec((1, _H), lambda i: (0, 0)),
        pl.BlockSpec((_H, _H), lambda i: (0, 0)),
    ],
    out_specs=pl.BlockSpec((_R, _H), lambda i: (i, 0)),
    out_shape=jax.ShapeDtypeStruct((_N, _H), jnp.float32),
)


def _tc3_body(aggp_ref, u2_ref, dinv_ref, b2_ref, batch_ref, sums_ref, cnts_ref):
    i = pl.program_id(0)
    out2 = (aggp_ref[0] + aggp_ref[1] + u2_ref[...]) * dinv_ref[...] + b2_ref[...]
    out2 = jnp.maximum(out2, 0.0)
    b = batch_ref[0, 0, :]
    gids = lax.broadcasted_iota(jnp.int32, (_G, _R), 0)
    onehot = (b[None, :] == gids).astype(jnp.float32)
    psums = jnp.dot(onehot, out2, preferred_element_type=jnp.float32)
    pcnts = jnp.sum(onehot, axis=1, keepdims=True)

    @pl.when(i == 0)
    def _():
        sums_ref[...] = psums
        cnts_ref[...] = pcnts

    @pl.when(i > 0)
    def _():
        sums_ref[...] += psums
        cnts_ref[...] += pcnts


_tc3 = pl.pallas_call(
    _tc3_body,
    grid=(_NB,),
    in_specs=[
        pl.BlockSpec((_NC, _R, _H), lambda i: (0, i, 0)),
        pl.BlockSpec((_R, _H), lambda i: (i, 0)),
        pl.BlockSpec((_R, 1), lambda i: (i, 0)),
        pl.BlockSpec((1, _H), lambda i: (0, 0)),
        pl.BlockSpec((1, 1, _R), lambda i: (i, 0, 0)),
    ],
    out_specs=[
        pl.BlockSpec((_G, _H), lambda i: (0, 0)),
        pl.BlockSpec((_G, 1), lambda i: (0, 0)),
    ],
    out_shape=[
        jax.ShapeDtypeStruct((_G, _H), jnp.float32),
        jax.ShapeDtypeStruct((_G, 1), jnp.float32),
    ],
)


def _tc4_body(sums_ref, cnts_ref, l1w_ref, l1b_ref, l2w_ref, l2b_ref,
              l3w_ref, l3b_ref, out_ref):
    g = sums_ref[...] / jnp.maximum(cnts_ref[...], 1.0)
    g = jnp.maximum(
        jnp.dot(g, l1w_ref[...], preferred_element_type=jnp.float32)
        + l1b_ref[...], 0.0)
    g = jnp.maximum(
        jnp.dot(g, l2w_ref[...], preferred_element_type=jnp.float32)
        + l2b_ref[...], 0.0)
    logits = (jnp.dot(g, l3w_ref[...], preferred_element_type=jnp.float32)
              + l3b_ref[...])
    m = jnp.max(logits, axis=-1, keepdims=True)
    lse = m + jnp.log(jnp.sum(jnp.exp(logits - m), axis=-1, keepdims=True))
    out_ref[...] = logits - lse


_tc4 = pl.pallas_call(
    _tc4_body,
    out_shape=jax.ShapeDtypeStruct((_G, _C), jnp.float32),
)


def kernel(x, edge_index, batch, W1, b1, W2, b2, lin1_W, lin1_b, lin2_W,
           lin2_b, lin3_W, lin3_b):
    src = edge_index[0]
    dst = edge_index[1]
    pad = _EPAD - _E
    scrap = _N + (jnp.arange(pad, dtype=jnp.int32) % (_NPAD - _N))
    srcp = jnp.concatenate([src, jnp.zeros((pad,), jnp.int32)]).reshape(_NWC, _CH)
    dstp = jnp.concatenate([dst, scrap]).reshape(_NWC, _CH)
    slack = jnp.zeros((_DG, 2, _CH), jnp.int32).at[:, 1, :].set(_N)
    earr = jnp.concatenate([jnp.stack([srcp, dstp], axis=1), slack], axis=0)

    degp = _sc_deg()(earr)
    u1, dinv = _tc1(x, degp, W1)
    agg1 = _sc_agg()(u1, earr)
    u2 = _tc2(agg1, u1, dinv, b1.reshape(1, _H), W2)
    agg2 = _sc_agg()(u2, earr)
    sums, cnts = _tc3(agg2, u2, dinv, b2.reshape(1, _H),
                      batch.reshape(_NB, 1, _R))
    out = _tc4(sums, cnts, lin1_W, lin1_b.reshape(1, _H), lin2_W,
               lin2_b.reshape(1, _H // 2), lin3_W, lin3_b.reshape(1, _C))
    return out


# speedup vs baseline: 1.4230x; 1.4141x over previous
import functools

import jax
import jax.numpy as jnp
from jax import lax
from jax.experimental import pallas as pl
from jax.experimental.pallas import tpu as pltpu
from jax.experimental.pallas import tpu_sc as plsc

_N = 10000
_E = 320000
_D = 128
_H = 64
_C = 10
_G = 64

_NC = 2
_NS = 16
_NW = _NC * _NS
_CH = 128
_CPW = 80
_NWC = _NW * _CPW
_EPAD = _NWC * _CH
_NPAD = 10240
_RPT = _NPAD // _NS
_HALF = _NPAD // 2
_SPT = _HALF // _NS
_DEGW = 16

_R = 1000
_NB = _N // _R

@functools.cache
def _mesh():
    return plsc.VectorSubcoreMesh(core_axis_name="c", subcore_axis_name="s",
                                  num_cores=_NC, num_subcores=_NS)


def _zero_shared_slice(zb, sh, row0):
    for k in range(5):
        pltpu.sync_copy(zb, sh.at[pl.ds(row0 + k * 128, 128)])


def _fill_const(ref, rows, width, value):
    def body(r, _):
        for k in range(width // 16):
            ref[r, pl.ds(k * 16, 16)] = jnp.full((16,), value, jnp.float32)
        return 0
    lax.fori_loop(0, rows, body, 0)


_DG = 8
_NGD = _CPW // _DG


def _sc_deg_body(e_hbm, out_hbm, deg_sh, ones_v, zb, idx_a, idx_b, ssem):
    c = lax.axis_index("c")
    s = lax.axis_index("s")
    wid = c * _NS + s
    _fill_const(ones_v, _CH, _DEGW, 1.0)
    _fill_const(zb, _CH, _DEGW, 0.0)
    _zero_shared_slice(zb, deg_sh, s * _RPT)
    plsc.subcore_barrier()
    base = wid * _CPW
    idx = (idx_a, idx_b)

    pltpu.sync_copy(e_hbm.at[pl.ds(base, _DG)], idx_a)

    def step2(k, _):
        for b in (0, 1):
            g = 2 * k + b
            p, pn = b, 1 - b

            @pl.when(g > 0)
            def _():
                for q in range(_DG):
                    pltpu.make_async_copy(
                        ones_v, deg_sh.at[idx[pn].at[q, 1]], ssem).wait()

            pltpu.sync_copy(e_hbm.at[pl.ds(base + (g + 1) * _DG, _DG)],
                            idx[pn])
            for q in range(_DG):
                pltpu.async_copy(ones_v, deg_sh.at[idx[p].at[q, 1]], ssem,
                                 add=True)
        return 0

    lax.fori_loop(0, _NGD // 2, step2, 0)
    for q in range(_DG):
        pltpu.make_async_copy(ones_v, deg_sh.at[idx_b.at[q, 1]], ssem).wait()
    plsc.subcore_barrier()
    pltpu.sync_copy(deg_sh.at[pl.ds(s * _RPT, _RPT)],
                    out_hbm.at[c, pl.ds(s * _RPT, _RPT)])


@functools.cache
def _sc_deg():
    return pl.kernel(
        _sc_deg_body,
        out_type=jax.ShapeDtypeStruct((_NC, _NPAD, _DEGW), jnp.float32),
        mesh=_mesh(),
        compiler_params=pltpu.CompilerParams(use_tc_tiling_on_sc=False),
        scratch_types=[
            pltpu.VMEM_SHARED((_NPAD, _DEGW), jnp.float32),
            pltpu.VMEM((_CH, _DEGW), jnp.float32),
            pltpu.VMEM((_CH, _DEGW), jnp.float32),
            pltpu.VMEM((_DG, 2, _CH), jnp.int32),
            pltpu.VMEM((_DG, 2, _CH), jnp.int32),
            pltpu.SemaphoreType.DMA,
        ],
    )


_GRP = 2
_CPA = _NWC // _NS
_NG = _CPA // _GRP


def _sc_agg_body(u_hbm, e_hbm, out_hbm, agg_sh, u_sh, zb, idx_a, idx_b,
                 lim_a, lim_b, rows_a, rows_b, gsem_a, gsem_b, ssem_a, ssem_b):
    c = lax.axis_index("c")
    s = lax.axis_index("s")
    _fill_const(zb, _CH, _H, 0.0)
    _zero_shared_slice(zb, agg_sh, s * _RPT)
    lo = c * _HALF
    pltpu.sync_copy(u_hbm.at[pl.ds(lo + s * _SPT, _SPT)],
                    u_sh.at[pl.ds(s * _SPT, _SPT)])

    @pl.when(s == 0)
    def _():
        pltpu.sync_copy(zb.at[pl.ds(0, 8)], u_sh.at[pl.ds(_HALF, 8)])

    plsc.subcore_barrier()
    base = s * _CPA
    idx = (idx_a, idx_b)
    lim = (lim_a, lim_b)
    rows = (rows_a, rows_b)
    gsem = (gsem_a, gsem_b)
    ssem = (ssem_a, ssem_b)

    def remap(p):
        for q in range(_GRP):
            for k in range(_CH // 16):
                v = idx[p][q, 0, pl.ds(k * 16, 16)]
                vl = v - lo
                inb = (vl >= 0) & (vl < _HALF)
                lim[p][q, pl.ds(k * 16, 16)] = jnp.where(inb, vl, _HALF)

    def fire_gathers(p):
        for q in range(_GRP):
            pltpu.async_copy(u_sh.at[lim[p].at[q]],
                             rows[p].at[pl.ds(q * _CH, _CH)], gsem[p])

    def wait_gathers(p):
        for q in range(_GRP):
            pltpu.make_async_copy(u_sh.at[lim[p].at[q]],
                                  rows[p].at[pl.ds(q * _CH, _CH)],
                                  gsem[p]).wait()

    def fire_scatters(p):
        for q in range(_GRP):
            pltpu.async_copy(rows[p].at[pl.ds(q * _CH, _CH)],
                             agg_sh.at[idx[p].at[q, 1]], ssem[p], add=True)

    def wait_scatters(p):
        for q in range(_GRP):
            pltpu.make_async_copy(rows[p].at[pl.ds(q * _CH, _CH)],
                                  agg_sh.at[idx[p].at[q, 1]], ssem[p]).wait()

    pltpu.sync_copy(e_hbm.at[pl.ds(base, _GRP)], idx_a)
    remap(0)
    fire_gathers(0)

    def step2(k, _):
        for b in (0, 1):
            g = 2 * k + b
            p, pn = b, 1 - b

            @pl.when(g > 0)
            def _():
                wait_scatters(pn)

            pltpu.sync_copy(e_hbm.at[pl.ds(base + (g + 1) * _GRP, _GRP)],
                            idx[pn])
            remap(pn)
            fire_gathers(pn)
            wait_gathers(p)
            fire_scatters(p)
        return 0

    lax.fori_loop(0, _NG // 2, step2, 0)
    wait_scatters(1)
    wait_gathers(0)
    plsc.subcore_barrier()
    pltpu.sync_copy(agg_sh.at[pl.ds(s * _RPT, _RPT)],
                    out_hbm.at[c, pl.ds(s * _RPT, _RPT)])


@functools.cache
def _sc_agg():
    return pl.kernel(
        _sc_agg_body,
        out_type=jax.ShapeDtypeStruct((_NC, _NPAD, _H), jnp.float32),
        mesh=_mesh(),
        compiler_params=pltpu.CompilerParams(use_tc_tiling_on_sc=False),
        scratch_types=[
            pltpu.VMEM_SHARED((_NPAD, _H), jnp.float32),
            pltpu.VMEM_SHARED((_HALF + 128, _H), jnp.float32),
            pltpu.VMEM((_CH, _H), jnp.float32),
            pltpu.VMEM((_GRP, 2, _CH), jnp.int32),
            pltpu.VMEM((_GRP, 2, _CH), jnp.int32),
            pltpu.VMEM((_GRP, _CH), jnp.int32),
            pltpu.VMEM((_GRP, _CH), jnp.int32),
            pltpu.VMEM((_GRP * _CH, _H), jnp.float32),
            pltpu.VMEM((_GRP * _CH, _H), jnp.float32),
            pltpu.SemaphoreType.DMA,
            pltpu.SemaphoreType.DMA,
            pltpu.SemaphoreType.DMA,
            pltpu.SemaphoreType.DMA,
        ],
    )


def _tc1_body(x_ref, degp_ref, w1_ref, u1_ref, dinv_ref):
    deg = degp_ref[0, :, 0:1] + degp_ref[1, :, 0:1] + 1.0
    dinv = lax.rsqrt(deg)
    h = jnp.dot(x_ref[...], w1_ref[...], preferred_element_type=jnp.float32)
    u1_ref[...] = h * dinv
    dinv_ref[...] = dinv


_tc1 = pl.pallas_call(
    _tc1_body,
    grid=(_NB,),
    in_specs=[
        pl.BlockSpec((_R, _D), lambda i: (i, 0)),
        pl.BlockSpec((_NC, _R, _DEGW), lambda i: (0, i, 0)),
        pl.BlockSpec((_D, _H), lambda i: (0, 0)),
    ],
    out_specs=[
        pl.BlockSpec((_R, _H), lambda i: (i, 0)),
        pl.BlockSpec((_R, 1), lambda i: (i, 0)),
    ],
    out_shape=[
        jax.ShapeDtypeStruct((_NPAD, _H), jnp.float32),
        jax.ShapeDtypeStruct((_N, 1), jnp.float32),
    ],
)


def _tc2_body(aggp_ref, u1_ref, dinv_ref, b1_ref, w2_ref, u2_ref):
    t = (aggp_ref[0] + aggp_ref[1] + u1_ref[...]) * dinv_ref[...] + b1_ref[...]
    t = jnp.maximum(t, 0.0)
    h2 = jnp.dot(t, w2_ref[...], preferred_element_type=jnp.float32)
    u2_ref[...] = h2 * dinv_ref[...]


_tc2 = pl.pallas_call(
    _tc2_body,
    grid=(_NB,),
    in_specs=[
        pl.BlockSpec((_NC, _R, _H), lambda i: (0, i, 0)),
        pl.BlockSpec((_R, _H), lambda i: (i, 0)),
        pl.BlockSpec((_R, 1), lambda i: (i, 0)),
        pl.BlockSpec((1, _H), lambda i: (0, 0)),
        pl.BlockSpec((_H, _H), lambda i: (0, 0)),
    ],
    out_specs=pl.BlockSpec((_R, _H), lambda i: (i, 0)),
    out_shape=jax.ShapeDtypeStruct((_NPAD, _H), jnp.float32),
)


def _tc3_body(aggp_ref, u2_ref, dinv_ref, b2_ref, batch_ref, sums_ref, cnts_ref):
    i = pl.program_id(0)
    out2 = (aggp_ref[0] + aggp_ref[1] + u2_ref[...]) * dinv_ref[...] + b2_ref[...]
    out2 = jnp.maximum(out2, 0.0)
    b = batch_ref[0, 0, :]
    gids = lax.broadcasted_iota(jnp.int32, (_G, _R), 0)
    onehot = (b[None, :] == gids).astype(jnp.float32)
    psums = jnp.dot(onehot, out2, preferred_element_type=jnp.float32)
    pcnts = jnp.sum(onehot, axis=1, keepdims=True)

    @pl.when(i == 0)
    def _():
        sums_ref[...] = psums
        cnts_ref[...] = pcnts

    @pl.when(i > 0)
    def _():
        sums_ref[...] += psums
        cnts_ref[...] += pcnts


_tc3 = pl.pallas_call(
    _tc3_body,
    grid=(_NB,),
    in_specs=[
        pl.BlockSpec((_NC, _R, _H), lambda i: (0, i, 0)),
        pl.BlockSpec((_R, _H), lambda i: (i, 0)),
        pl.BlockSpec((_R, 1), lambda i: (i, 0)),
        pl.BlockSpec((1, _H), lambda i: (0, 0)),
        pl.BlockSpec((1, 1, _R), lambda i: (i, 0, 0)),
    ],
    out_specs=[
        pl.BlockSpec((_G, _H), lambda i: (0, 0)),
        pl.BlockSpec((_G, 1), lambda i: (0, 0)),
    ],
    out_shape=[
        jax.ShapeDtypeStruct((_G, _H), jnp.float32),
        jax.ShapeDtypeStruct((_G, 1), jnp.float32),
    ],
)


def _tc4_body(sums_ref, cnts_ref, l1w_ref, l1b_ref, l2w_ref, l2b_ref,
              l3w_ref, l3b_ref, out_ref):
    g = sums_ref[...] / jnp.maximum(cnts_ref[...], 1.0)
    g = jnp.maximum(
        jnp.dot(g, l1w_ref[...], preferred_element_type=jnp.float32)
        + l1b_ref[...], 0.0)
    g = jnp.maximum(
        jnp.dot(g, l2w_ref[...], preferred_element_type=jnp.float32)
        + l2b_ref[...], 0.0)
    logits = (jnp.dot(g, l3w_ref[...], preferred_element_type=jnp.float32)
              + l3b_ref[...])
    m = jnp.max(logits, axis=-1, keepdims=True)
    lse = m + jnp.log(jnp.sum(jnp.exp(logits - m), axis=-1, keepdims=True))
    out_ref[...] = logits - lse


_tc4 = pl.pallas_call(
    _tc4_body,
    out_shape=jax.ShapeDtypeStruct((_G, _C), jnp.float32),
)


def kernel(x, edge_index, batch, W1, b1, W2, b2, lin1_W, lin1_b, lin2_W,
           lin2_b, lin3_W, lin3_b):
    src = edge_index[0]
    dst = edge_index[1]
    pad = _EPAD - _E
    scrap = _N + (jnp.arange(pad, dtype=jnp.int32) % (_NPAD - _N))
    srcp = jnp.concatenate([src, jnp.zeros((pad,), jnp.int32)]).reshape(_NWC, _CH)
    dstp = jnp.concatenate([dst, scrap]).reshape(_NWC, _CH)
    slack = jnp.zeros((_DG, 2, _CH), jnp.int32).at[:, 1, :].set(_N)
    earr = jnp.concatenate([jnp.stack([srcp, dstp], axis=1), slack], axis=0)

    degp = _sc_deg()(earr)
    u1, dinv = _tc1(x, degp, W1)
    agg1 = _sc_agg()(u1, earr)
    u2 = _tc2(agg1, u1, dinv, b1.reshape(1, _H), W2)
    agg2 = _sc_agg()(u2, earr)
    sums, cnts = _tc3(agg2, u2, dinv, b2.reshape(1, _H),
                      batch.reshape(_NB, 1, _R))
    out = _tc4(sums, cnts, lin1_W, lin1_b.reshape(1, _H), lin2_W,
               lin2_b.reshape(1, _H // 2), lin3_W, lin3_b.reshape(1, _C))
    return out


# speedup vs baseline: 1.9577x; 1.3758x over previous
import functools

import jax
import jax.numpy as jnp
from jax import lax
from jax.experimental import pallas as pl
from jax.experimental.pallas import tpu as pltpu
from jax.experimental.pallas import tpu_sc as plsc

_N = 10000
_E = 320000
_D = 128
_H = 64
_HW = _H // 2
_C = 10
_G = 64

_NC = 2
_NS = 16
_CH = 128
_NWC = 2560
_EPAD = _NWC * _CH
_NPAD = 10240
_RPT = _NPAD // _NS
_DEGW = 16

_R = 1000
_NB = _N // _R


@functools.cache
def _mesh():
    return plsc.VectorSubcoreMesh(core_axis_name="c", subcore_axis_name="s",
                                  num_cores=_NC, num_subcores=_NS)


def _zero_shared_slice(zb, sh, row0):
    for k in range(5):
        pltpu.sync_copy(zb, sh.at[pl.ds(row0 + k * 128, 128)])


def _fill_const(ref, rows, width, value):
    def body(r, _):
        for k in range(width // 16):
            ref[r, pl.ds(k * 16, 16)] = jnp.full((16,), value, jnp.float32)
        return 0
    lax.fori_loop(0, rows, body, 0)


_DG = 8
_CPW = _NWC // (_NC * _NS)
_NGD = _CPW // _DG


def _sc_deg_body(d_hbm, out_hbm, deg_sh, ones_v, zb, idx_a, idx_b, ssem):
    c = lax.axis_index("c")
    s = lax.axis_index("s")
    wid = c * _NS + s
    _fill_const(ones_v, _CH, _DEGW, 1.0)
    _fill_const(zb, _CH, _DEGW, 0.0)
    _zero_shared_slice(zb, deg_sh, s * _RPT)
    plsc.subcore_barrier()
    base = wid * _CPW
    idx = (idx_a, idx_b)

    pltpu.sync_copy(d_hbm.at[pl.ds(base, _DG)], idx_a)

    def step2(k, _):
        for b in (0, 1):
            g = 2 * k + b
            p, pn = b, 1 - b

            @pl.when(g > 0)
            def _():
                for q in range(_DG):
                    pltpu.make_async_copy(
                        ones_v, deg_sh.at[idx[pn].at[q]], ssem).wait()

            pltpu.sync_copy(d_hbm.at[pl.ds(base + (g + 1) * _DG, _DG)],
                            idx[pn])
            for q in range(_DG):
                pltpu.async_copy(ones_v, deg_sh.at[idx[p].at[q]], ssem,
                                 add=True)
        return 0

    lax.fori_loop(0, _NGD // 2, step2, 0)
    for q in range(_DG):
        pltpu.make_async_copy(ones_v, deg_sh.at[idx_b.at[q]], ssem).wait()
    plsc.subcore_barrier()
    pltpu.sync_copy(deg_sh.at[pl.ds(s * _RPT, _RPT)],
                    out_hbm.at[c, pl.ds(s * _RPT, _RPT)])


@functools.cache
def _sc_deg():
    return pl.kernel(
        _sc_deg_body,
        out_type=jax.ShapeDtypeStruct((_NC, _NPAD, _DEGW), jnp.float32),
        mesh=_mesh(),
        compiler_params=pltpu.CompilerParams(use_tc_tiling_on_sc=False),
        scratch_types=[
            pltpu.VMEM_SHARED((_NPAD, _DEGW), jnp.float32),
            pltpu.VMEM((_CH, _DEGW), jnp.float32),
            pltpu.VMEM((_CH, _DEGW), jnp.float32),
            pltpu.VMEM((_DG, _CH), jnp.int32),
            pltpu.VMEM((_DG, _CH), jnp.int32),
            pltpu.SemaphoreType.DMA,
        ],
    )


_GRP = 4
_CPA = _NWC // _NS
_NG = _CPA // _GRP


def _sc_agg_body(u_hbm, s_hbm, d_hbm, out_hbm, agg_sh, u_sh, zb, sidx_a,
                 sidx_b, didx_a, didx_b, rows_a, rows_b, gsem_a, gsem_b,
                 ssem_a, ssem_b):
    c = lax.axis_index("c")
    s = lax.axis_index("s")
    _fill_const(zb, _CH, _HW, 0.0)
    _zero_shared_slice(zb, agg_sh, s * _RPT)
    pltpu.sync_copy(u_hbm.at[c, pl.ds(s * _RPT, _RPT)],
                    u_sh.at[pl.ds(s * _RPT, _RPT)])
    plsc.subcore_barrier()
    base = s * _CPA
    sidx = (sidx_a, sidx_b)
    didx = (didx_a, didx_b)
    rows = (rows_a, rows_b)
    gsem = (gsem_a, gsem_b)
    ssem = (ssem_a, ssem_b)

    def load_idx(p, g):
        pltpu.sync_copy(s_hbm.at[pl.ds(base + g * _GRP, _GRP)], sidx[p])
        pltpu.sync_copy(d_hbm.at[pl.ds(base + g * _GRP, _GRP)], didx[p])

    def fire_gathers(p):
        for q in range(_GRP):
            pltpu.async_copy(u_sh.at[sidx[p].at[q]],
                             rows[p].at[pl.ds(q * _CH, _CH)], gsem[p])

    def wait_gathers(p):
        for q in range(_GRP):
            pltpu.make_async_copy(u_sh.at[sidx[p].at[q]],
                                  rows[p].at[pl.ds(q * _CH, _CH)],
                                  gsem[p]).wait()

    def fire_scatters(p):
        for q in range(_GRP):
            pltpu.async_copy(rows[p].at[pl.ds(q * _CH, _CH)],
                             agg_sh.at[didx[p].at[q]], ssem[p], add=True)

    def wait_scatters(p):
        for q in range(_GRP):
            pltpu.make_async_copy(rows[p].at[pl.ds(q * _CH, _CH)],
                                  agg_sh.at[didx[p].at[q]], ssem[p]).wait()

    load_idx(0, 0)
    fire_gathers(0)

    def step2(k, _):
        for b in (0, 1):
            g = 2 * k + b
            p, pn = b, 1 - b

            @pl.when(g > 0)
            def _():
                wait_scatters(pn)

            load_idx(pn, g + 1)
            fire_gathers(pn)
            wait_gathers(p)
            fire_scatters(p)
        return 0

    lax.fori_loop(0, _NG // 2, step2, 0)
    wait_scatters(1)
    wait_gathers(0)
    plsc.subcore_barrier()
    pltpu.sync_copy(agg_sh.at[pl.ds(s * _RPT, _RPT)],
                    out_hbm.at[c, pl.ds(s * _RPT, _RPT)])


@functools.cache
def _sc_agg():
    return pl.kernel(
        _sc_agg_body,
        out_type=jax.ShapeDtypeStruct((_NC, _NPAD, _HW), jnp.float32),
        mesh=_mesh(),
        compiler_params=pltpu.CompilerParams(use_tc_tiling_on_sc=False),
        scratch_types=[
            pltpu.VMEM_SHARED((_NPAD, _HW), jnp.float32),
            pltpu.VMEM_SHARED((_NPAD, _HW), jnp.float32),
            pltpu.VMEM((_CH, _HW), jnp.float32),
            pltpu.VMEM((_GRP, _CH), jnp.int32),
            pltpu.VMEM((_GRP, _CH), jnp.int32),
            pltpu.VMEM((_GRP, _CH), jnp.int32),
            pltpu.VMEM((_GRP, _CH), jnp.int32),
            pltpu.VMEM((_GRP * _CH, _HW), jnp.float32),
            pltpu.VMEM((_GRP * _CH, _HW), jnp.float32),
            pltpu.SemaphoreType.DMA,
            pltpu.SemaphoreType.DMA,
            pltpu.SemaphoreType.DMA,
            pltpu.SemaphoreType.DMA,
        ],
    )


def _tc1_body(x_ref, degp_ref, w1_ref, u1_ref, dinv_ref):
    deg = degp_ref[0, :, 0:1] + degp_ref[1, :, 0:1] + 1.0
    dinv = lax.rsqrt(deg)
    h = jnp.dot(x_ref[...], w1_ref[...], preferred_element_type=jnp.float32)
    u = h * dinv
    u1_ref[0] = u[:, :_HW]
    u1_ref[1] = u[:, _HW:]
    dinv_ref[...] = dinv


_tc1 = pl.pallas_call(
    _tc1_body,
    grid=(_NB,),
    in_specs=[
        pl.BlockSpec((_R, _D), lambda i: (i, 0)),
        pl.BlockSpec((_NC, _R, _DEGW), lambda i: (0, i, 0)),
        pl.BlockSpec((_D, _H), lambda i: (0, 0)),
    ],
    out_specs=[
        pl.BlockSpec((_NC, _R, _HW), lambda i: (0, i, 0)),
        pl.BlockSpec((_R, 1), lambda i: (i, 0)),
    ],
    out_shape=[
        jax.ShapeDtypeStruct((_NC, _NPAD, _HW), jnp.float32),
        jax.ShapeDtypeStruct((_N, 1), jnp.float32),
    ],
)


def _tc2_body(aggp_ref, u1_ref, dinv_ref, b1_ref, w2_ref, u2_ref):
    aggu = jnp.concatenate([aggp_ref[0] + u1_ref[0], aggp_ref[1] + u1_ref[1]],
                           axis=1)
    t = aggu * dinv_ref[...] + b1_ref[...]
    t = jnp.maximum(t, 0.0)
    h2 = jnp.dot(t, w2_ref[...], preferred_element_type=jnp.float32)
    u = h2 * dinv_ref[...]
    u2_ref[0] = u[:, :_HW]
    u2_ref[1] = u[:, _HW:]


_tc2 = pl.pallas_call(
    _tc2_body,
    grid=(_NB,),
    in_specs=[
        pl.BlockSpec((_NC, _R, _HW), lambda i: (0, i, 0)),
        pl.BlockSpec((_NC, _R, _HW), lambda i: (0, i, 0)),
        pl.BlockSpec((_R, 1), lambda i: (i, 0)),
        pl.BlockSpec((1, _H), lambda i: (0, 0)),
        pl.BlockSpec((_H, _H), lambda i: (0, 0)),
    ],
    out_specs=pl.BlockSpec((_NC, _R, _HW), lambda i: (0, i, 0)),
    out_shape=jax.ShapeDtypeStruct((_NC, _NPAD, _HW), jnp.float32),
)


def _tc3_body(aggp_ref, u2_ref, dinv_ref, b2_ref, batch_ref, sums_ref,
              cnts_ref):
    i = pl.program_id(0)
    aggu = jnp.concatenate([aggp_ref[0] + u2_ref[0], aggp_ref[1] + u2_ref[1]],
                           axis=1)
    out2 = aggu * dinv_ref[...] + b2_ref[...]
    out2 = jnp.maximum(out2, 0.0)
    b = batch_ref[0, 0, :]
    gids = lax.broadcasted_iota(jnp.int32, (_G, _R), 0)
    onehot = (b[None, :] == gids).astype(jnp.float32)
    psums = jnp.dot(onehot, out2, preferred_element_type=jnp.float32)
    pcnts = jnp.sum(onehot, axis=1, keepdims=True)

    @pl.when(i == 0)
    def _():
        sums_ref[...] = psums
        cnts_ref[...] = pcnts

    @pl.when(i > 0)
    def _():
        sums_ref[...] += psums
        cnts_ref[...] += pcnts


_tc3 = pl.pallas_call(
    _tc3_body,
    grid=(_NB,),
    in_specs=[
        pl.BlockSpec((_NC, _R, _HW), lambda i: (0, i, 0)),
        pl.BlockSpec((_NC, _R, _HW), lambda i: (0, i, 0)),
        pl.BlockSpec((_R, 1), lambda i: (i, 0)),
        pl.BlockSpec((1, _H), lambda i: (0, 0)),
        pl.BlockSpec((1, 1, _R), lambda i: (i, 0, 0)),
    ],
    out_specs=[
        pl.BlockSpec((_G, _H), lambda i: (0, 0)),
        pl.BlockSpec((_G, 1), lambda i: (0, 0)),
    ],
    out_shape=[
        jax.ShapeDtypeStruct((_G, _H), jnp.float32),
        jax.ShapeDtypeStruct((_G, 1), jnp.float32),
    ],
)


def _tc4_body(sums_ref, cnts_ref, l1w_ref, l1b_ref, l2w_ref, l2b_ref,
              l3w_ref, l3b_ref, out_ref):
    g = sums_ref[...] / jnp.maximum(cnts_ref[...], 1.0)
    g = jnp.maximum(
        jnp.dot(g, l1w_ref[...], preferred_element_type=jnp.float32)
        + l1b_ref[...], 0.0)
    g = jnp.maximum(
        jnp.dot(g, l2w_ref[...], preferred_element_type=jnp.float32)
        + l2b_ref[...], 0.0)
    logits = (jnp.dot(g, l3w_ref[...], preferred_element_type=jnp.float32)
              + l3b_ref[...])
    m = jnp.max(logits, axis=-1, keepdims=True)
    lse = m + jnp.log(jnp.sum(jnp.exp(logits - m), axis=-1, keepdims=True))
    out_ref[...] = logits - lse


_tc4 = pl.pallas_call(
    _tc4_body,
    out_shape=jax.ShapeDtypeStruct((_G, _C), jnp.float32),
)


def kernel(x, edge_index, batch, W1, b1, W2, b2, lin1_W, lin1_b, lin2_W,
           lin2_b, lin3_W, lin3_b):
    src = edge_index[0]
    dst = edge_index[1]
    pad = _EPAD - _E + _DG * _CH
    scrap = _N + (jnp.arange(pad, dtype=jnp.int32) % (_NPAD - _N))
    srcp = jnp.concatenate([src, jnp.zeros((pad,), jnp.int32)]).reshape(-1, _CH)
    dstp = jnp.concatenate([dst, scrap]).reshape(-1, _CH)

    degp = _sc_deg()(dstp)
    u1, dinv = _tc1(x, degp, W1)
    agg1 = _sc_agg()(u1, srcp, dstp)
    u2 = _tc2(agg1, u1, dinv, b1.reshape(1, _H), W2)
    agg2 = _sc_agg()(u2, srcp, dstp)
    sums, cnts = _tc3(agg2, u2, dinv, b2.reshape(1, _H),
                      batch.reshape(_NB, 1, _R))
    out = _tc4(sums, cnts, lin1_W, lin1_b.reshape(1, _H), lin2_W,
               lin2_b.reshape(1, _H // 2), lin3_W, lin3_b.reshape(1, _C))
    return out


# speedup vs baseline: 2.1590x; 1.1028x over previous
import functools

import jax
import jax.numpy as jnp
from jax import lax
from jax.experimental import pallas as pl
from jax.experimental.pallas import tpu as pltpu
from jax.experimental.pallas import tpu_sc as plsc

_N = 10000
_E = 320000
_D = 128
_H = 64
_HW = _H // 2
_C = 10
_G = 64

_NC = 2
_NS = 16
_CH = 128
_NWC = 2560
_EPAD = _NWC * _CH
_NPAD = 10240
_RPT = _NPAD // _NS
_DEGW = 16

_R = 1000
_NB = _N // _R


@functools.cache
def _mesh():
    return plsc.VectorSubcoreMesh(core_axis_name="c", subcore_axis_name="s",
                                  num_cores=_NC, num_subcores=_NS)


def _zero_shared_slice(zb, sh, row0):
    for k in range(5):
        pltpu.sync_copy(zb, sh.at[pl.ds(row0 + k * 128, 128)])


def _fill_const(ref, rows, width, value):
    def body(r, _):
        for k in range(width // 16):
            ref[r, pl.ds(k * 16, 16)] = jnp.full((16,), value, jnp.float32)
        return 0
    lax.fori_loop(0, rows, body, 0)


_DG = 8
_CPW = _NWC // (_NC * _NS)
_NGD = _CPW // _DG


def _sc_deg_body(d_hbm, out_hbm, deg_sh, ones_v, zb, idx_a, idx_b, ssem):
    c = lax.axis_index("c")
    s = lax.axis_index("s")
    wid = c * _NS + s
    _fill_const(ones_v, _CH, _DEGW, 1.0)
    _fill_const(zb, _CH, _DEGW, 0.0)
    _zero_shared_slice(zb, deg_sh, s * _RPT)
    plsc.subcore_barrier()
    base = wid * _CPW
    idx = (idx_a, idx_b)

    pltpu.sync_copy(d_hbm.at[pl.ds(base, _DG)], idx_a)

    def step2(k, _):
        for b in (0, 1):
            g = 2 * k + b
            p, pn = b, 1 - b

            @pl.when(g > 0)
            def _():
                for q in range(_DG):
                    pltpu.make_async_copy(
                        ones_v, deg_sh.at[idx[pn].at[q]], ssem).wait()

            pltpu.sync_copy(d_hbm.at[pl.ds(base + (g + 1) * _DG, _DG)],
                            idx[pn])
            for q in range(_DG):
                pltpu.async_copy(ones_v, deg_sh.at[idx[p].at[q]], ssem,
                                 add=True)
        return 0

    lax.fori_loop(0, _NGD // 2, step2, 0)
    for q in range(_DG):
        pltpu.make_async_copy(ones_v, deg_sh.at[idx_b.at[q]], ssem).wait()
    plsc.subcore_barrier()
    pltpu.sync_copy(deg_sh.at[pl.ds(s * _RPT, _RPT)],
                    out_hbm.at[c, pl.ds(s * _RPT, _RPT)])


@functools.cache
def _sc_deg():
    return pl.kernel(
        _sc_deg_body,
        out_type=jax.ShapeDtypeStruct((_NC, _NPAD, _DEGW), jnp.float32),
        mesh=_mesh(),
        compiler_params=pltpu.CompilerParams(use_tc_tiling_on_sc=False),
        scratch_types=[
            pltpu.VMEM_SHARED((_NPAD, _DEGW), jnp.float32),
            pltpu.VMEM((_CH, _DEGW), jnp.float32),
            pltpu.VMEM((_CH, _DEGW), jnp.float32),
            pltpu.VMEM((_DG, _CH), jnp.int32),
            pltpu.VMEM((_DG, _CH), jnp.int32),
            pltpu.SemaphoreType.DMA,
        ],
    )


_GRP = 8
_CPA = _NWC // _NS
_NG = _CPA // _GRP


def _sc_agg_body(u_hbm, s_hbm, d_hbm, out_hbm, agg_sh, u_sh, zb, sidx_a,
                 sidx_b, didx_a, didx_b, rows_a, rows_b, gsem_a, gsem_b,
                 ssem_a, ssem_b):
    c = lax.axis_index("c")
    s = lax.axis_index("s")
    _fill_const(zb, _CH, _HW, 0.0)
    _zero_shared_slice(zb, agg_sh, s * _RPT)
    pltpu.sync_copy(u_hbm.at[c, pl.ds(s * _RPT, _RPT)],
                    u_sh.at[pl.ds(s * _RPT, _RPT)])
    plsc.subcore_barrier()
    base = s * _CPA
    sidx = (sidx_a, sidx_b)
    didx = (didx_a, didx_b)
    rows = (rows_a, rows_b)
    gsem = (gsem_a, gsem_b)
    ssem = (ssem_a, ssem_b)

    def load_idx(p, g):
        pltpu.sync_copy(s_hbm.at[pl.ds(base + g * _GRP, _GRP)], sidx[p])
        pltpu.sync_copy(d_hbm.at[pl.ds(base + g * _GRP, _GRP)], didx[p])

    def fire_gathers(p):
        for q in range(_GRP):
            pltpu.async_copy(u_sh.at[sidx[p].at[q]],
                             rows[p].at[pl.ds(q * _CH, _CH)], gsem[p])

    def wait_gathers(p):
        for q in range(_GRP):
            pltpu.make_async_copy(u_sh.at[sidx[p].at[q]],
                                  rows[p].at[pl.ds(q * _CH, _CH)],
                                  gsem[p]).wait()

    def fire_scatters(p):
        for q in range(_GRP):
            pltpu.async_copy(rows[p].at[pl.ds(q * _CH, _CH)],
                             agg_sh.at[didx[p].at[q]], ssem[p], add=True)

    def wait_scatters(p):
        for q in range(_GRP):
            pltpu.make_async_copy(rows[p].at[pl.ds(q * _CH, _CH)],
                                  agg_sh.at[didx[p].at[q]], ssem[p]).wait()

    load_idx(0, 0)
    fire_gathers(0)

    def step2(k, _):
        for b in (0, 1):
            g = 2 * k + b
            p, pn = b, 1 - b

            @pl.when(g > 0)
            def _():
                wait_scatters(pn)

            load_idx(pn, g + 1)
            fire_gathers(pn)
            wait_gathers(p)
            fire_scatters(p)
        return 0

    lax.fori_loop(0, _NG // 2, step2, 0)
    wait_scatters(1)
    wait_gathers(0)
    plsc.subcore_barrier()
    pltpu.sync_copy(agg_sh.at[pl.ds(s * _RPT, _RPT)],
                    out_hbm.at[c, pl.ds(s * _RPT, _RPT)])


@functools.cache
def _sc_agg():
    return pl.kernel(
        _sc_agg_body,
        out_type=jax.ShapeDtypeStruct((_NC, _NPAD, _HW), jnp.float32),
        mesh=_mesh(),
        compiler_params=pltpu.CompilerParams(use_tc_tiling_on_sc=False),
        scratch_types=[
            pltpu.VMEM_SHARED((_NPAD, _HW), jnp.float32),
            pltpu.VMEM_SHARED((_NPAD, _HW), jnp.float32),
            pltpu.VMEM((_CH, _HW), jnp.float32),
            pltpu.VMEM((_GRP, _CH), jnp.int32),
            pltpu.VMEM((_GRP, _CH), jnp.int32),
            pltpu.VMEM((_GRP, _CH), jnp.int32),
            pltpu.VMEM((_GRP, _CH), jnp.int32),
            pltpu.VMEM((_GRP * _CH, _HW), jnp.float32),
            pltpu.VMEM((_GRP * _CH, _HW), jnp.float32),
            pltpu.SemaphoreType.DMA,
            pltpu.SemaphoreType.DMA,
            pltpu.SemaphoreType.DMA,
            pltpu.SemaphoreType.DMA,
        ],
    )


def _tc1_body(x_ref, degp_ref, w1_ref, u1_ref, dinv_ref):
    deg = degp_ref[0, :, 0:1] + degp_ref[1, :, 0:1] + 1.0
    dinv = lax.rsqrt(deg)
    h = jnp.dot(x_ref[...], w1_ref[...], preferred_element_type=jnp.float32)
    u = h * dinv
    u1_ref[0] = u[:, :_HW]
    u1_ref[1] = u[:, _HW:]
    dinv_ref[...] = dinv


_tc1 = pl.pallas_call(
    _tc1_body,
    grid=(_NB,),
    in_specs=[
        pl.BlockSpec((_R, _D), lambda i: (i, 0)),
        pl.BlockSpec((_NC, _R, _DEGW), lambda i: (0, i, 0)),
        pl.BlockSpec((_D, _H), lambda i: (0, 0)),
    ],
    out_specs=[
        pl.BlockSpec((_NC, _R, _HW), lambda i: (0, i, 0)),
        pl.BlockSpec((_R, 1), lambda i: (i, 0)),
    ],
    out_shape=[
        jax.ShapeDtypeStruct((_NC, _NPAD, _HW), jnp.float32),
        jax.ShapeDtypeStruct((_N, 1), jnp.float32),
    ],
)


def _tc2_body(aggp_ref, u1_ref, dinv_ref, b1_ref, w2_ref, u2_ref):
    aggu = jnp.concatenate([aggp_ref[0] + u1_ref[0], aggp_ref[1] + u1_ref[1]],
                           axis=1)
    t = aggu * dinv_ref[...] + b1_ref[...]
    t = jnp.maximum(t, 0.0)
    h2 = jnp.dot(t, w2_ref[...], preferred_element_type=jnp.float32)
    u = h2 * dinv_ref[...]
    u2_ref[0] = u[:, :_HW]
    u2_ref[1] = u[:, _HW:]


_tc2 = pl.pallas_call(
    _tc2_body,
    grid=(_NB,),
    in_specs=[
        pl.BlockSpec((_NC, _R, _HW), lambda i: (0, i, 0)),
        pl.BlockSpec((_NC, _R, _HW), lambda i: (0, i, 0)),
        pl.BlockSpec((_R, 1), lambda i: (i, 0)),
        pl.BlockSpec((1, _H), lambda i: (0, 0)),
        pl.BlockSpec((_H, _H), lambda i: (0, 0)),
    ],
    out_specs=pl.BlockSpec((_NC, _R, _HW), lambda i: (0, i, 0)),
    out_shape=jax.ShapeDtypeStruct((_NC, _NPAD, _HW), jnp.float32),
)


def _tc3_body(aggp_ref, u2_ref, dinv_ref, b2_ref, batch_ref, sums_ref,
              cnts_ref):
    i = pl.program_id(0)
    aggu = jnp.concatenate([aggp_ref[0] + u2_ref[0], aggp_ref[1] + u2_ref[1]],
                           axis=1)
    out2 = aggu * dinv_ref[...] + b2_ref[...]
    out2 = jnp.maximum(out2, 0.0)
    b = batch_ref[0, 0, :]
    gids = lax.broadcasted_iota(jnp.int32, (_G, _R), 0)
    onehot = (b[None, :] == gids).astype(jnp.float32)
    psums = jnp.dot(onehot, out2, preferred_element_type=jnp.float32)
    pcnts = jnp.sum(onehot, axis=1, keepdims=True)

    @pl.when(i == 0)
    def _():
        sums_ref[...] = psums
        cnts_ref[...] = pcnts

    @pl.when(i > 0)
    def _():
        sums_ref[...] += psums
        cnts_ref[...] += pcnts


_tc3 = pl.pallas_call(
    _tc3_body,
    grid=(_NB,),
    in_specs=[
        pl.BlockSpec((_NC, _R, _HW), lambda i: (0, i, 0)),
        pl.BlockSpec((_NC, _R, _HW), lambda i: (0, i, 0)),
        pl.BlockSpec((_R, 1), lambda i: (i, 0)),
        pl.BlockSpec((1, _H), lambda i: (0, 0)),
        pl.BlockSpec((1, 1, _R), lambda i: (i, 0, 0)),
    ],
    out_specs=[
        pl.BlockSpec((_G, _H), lambda i: (0, 0)),
        pl.BlockSpec((_G, 1), lambda i: (0, 0)),
    ],
    out_shape=[
        jax.ShapeDtypeStruct((_G, _H), jnp.float32),
        jax.ShapeDtypeStruct((_G, 1), jnp.float32),
    ],
)


def _tc4_body(sums_ref, cnts_ref, l1w_ref, l1b_ref, l2w_ref, l2b_ref,
              l3w_ref, l3b_ref, out_ref):
    g = sums_ref[...] / jnp.maximum(cnts_ref[...], 1.0)
    g = jnp.maximum(
        jnp.dot(g, l1w_ref[...], preferred_element_type=jnp.float32)
        + l1b_ref[...], 0.0)
    g = jnp.maximum(
        jnp.dot(g, l2w_ref[...], preferred_element_type=jnp.float32)
        + l2b_ref[...], 0.0)
    logits = (jnp.dot(g, l3w_ref[...], preferred_element_type=jnp.float32)
              + l3b_ref[...])
    m = jnp.max(logits, axis=-1, keepdims=True)
    lse = m + jnp.log(jnp.sum(jnp.exp(logits - m), axis=-1, keepdims=True))
    out_ref[...] = logits - lse


_tc4 = pl.pallas_call(
    _tc4_body,
    out_shape=jax.ShapeDtypeStruct((_G, _C), jnp.float32),
)


def kernel(x, edge_index, batch, W1, b1, W2, b2, lin1_W, lin1_b, lin2_W,
           lin2_b, lin3_W, lin3_b):
    src = edge_index[0]
    dst = edge_index[1]
    pad = _EPAD - _E + _DG * _CH
    scrap = _N + (jnp.arange(pad, dtype=jnp.int32) % (_NPAD - _N))
    srcp = jnp.concatenate([src, jnp.zeros((pad,), jnp.int32)]).reshape(-1, _CH)
    dstp = jnp.concatenate([dst, scrap]).reshape(-1, _CH)

    degp = _sc_deg()(dstp)
    u1, dinv = _tc1(x, degp, W1)
    agg1 = _sc_agg()(u1, srcp, dstp)
    u2 = _tc2(agg1, u1, dinv, b1.reshape(1, _H), W2)
    agg2 = _sc_agg()(u2, srcp, dstp)
    sums, cnts = _tc3(agg2, u2, dinv, b2.reshape(1, _H),
                      batch.reshape(_NB, 1, _R))
    out = _tc4(sums, cnts, lin1_W, lin1_b.reshape(1, _H), lin2_W,
               lin2_b.reshape(1, _H // 2), lin3_W, lin3_b.reshape(1, _C))
    return out


# speedup vs baseline: 2.1839x; 1.0115x over previous
import functools

import jax
import jax.numpy as jnp
from jax import lax
from jax.experimental import pallas as pl
from jax.experimental.pallas import tpu as pltpu
from jax.experimental.pallas import tpu_sc as plsc

_N = 10000
_E = 320000
_D = 128
_H = 64
_HW = _H // 2
_C = 10
_G = 64

_NC = 2
_NS = 16
_CH = 128
_NWC = 2560
_EPAD = _NWC * _CH
_NPAD = 10240
_RPT = _NPAD // _NS
_DEGW = 16

_R = 2000
_NB = _N // _R


@functools.cache
def _mesh():
    return plsc.VectorSubcoreMesh(core_axis_name="c", subcore_axis_name="s",
                                  num_cores=_NC, num_subcores=_NS)


def _zero_shared_slice(zb, sh, row0):
    for k in range(5):
        pltpu.sync_copy(zb, sh.at[pl.ds(row0 + k * 128, 128)])


def _fill_const(ref, rows, width, value):
    def body(r, _):
        for k in range(width // 16):
            ref[r, pl.ds(k * 16, 16)] = jnp.full((16,), value, jnp.float32)
        return 0
    lax.fori_loop(0, rows, body, 0)


_DG = 8
_CPW = _NWC // (_NC * _NS)
_NGD = _CPW // _DG


def _sc_deg_body(d_hbm, out_hbm, deg_sh, ones_v, zb, idx_a, idx_b, ssem):
    c = lax.axis_index("c")
    s = lax.axis_index("s")
    wid = c * _NS + s
    _fill_const(ones_v, _CH, _DEGW, 1.0)
    _fill_const(zb, _CH, _DEGW, 0.0)
    _zero_shared_slice(zb, deg_sh, s * _RPT)
    plsc.subcore_barrier()
    base = wid * _CPW
    idx = (idx_a, idx_b)

    pltpu.sync_copy(d_hbm.at[pl.ds(base, _DG)], idx_a)

    def step2(k, _):
        for b in (0, 1):
            g = 2 * k + b
            p, pn = b, 1 - b

            @pl.when(g > 0)
            def _():
                for q in range(_DG):
                    pltpu.make_async_copy(
                        ones_v, deg_sh.at[idx[pn].at[q]], ssem).wait()

            pltpu.sync_copy(d_hbm.at[pl.ds(base + (g + 1) * _DG, _DG)],
                            idx[pn])
            for q in range(_DG):
                pltpu.async_copy(ones_v, deg_sh.at[idx[p].at[q]], ssem,
                                 add=True)
        return 0

    lax.fori_loop(0, _NGD // 2, step2, 0)
    for q in range(_DG):
        pltpu.make_async_copy(ones_v, deg_sh.at[idx_b.at[q]], ssem).wait()
    plsc.subcore_barrier()
    pltpu.sync_copy(deg_sh.at[pl.ds(s * _RPT, _RPT), pl.ds(0, 8)],
                    out_hbm.at[c, pl.ds(s * _RPT, _RPT)])


@functools.cache
def _sc_deg():
    return pl.kernel(
        _sc_deg_body,
        out_type=jax.ShapeDtypeStruct((_NC, _NPAD, 8), jnp.float32),
        mesh=_mesh(),
        compiler_params=pltpu.CompilerParams(use_tc_tiling_on_sc=False),
        scratch_types=[
            pltpu.VMEM_SHARED((_NPAD, _DEGW), jnp.float32),
            pltpu.VMEM((_CH, _DEGW), jnp.float32),
            pltpu.VMEM((_CH, _DEGW), jnp.float32),
            pltpu.VMEM((_DG, _CH), jnp.int32),
            pltpu.VMEM((_DG, _CH), jnp.int32),
            pltpu.SemaphoreType.DMA,
        ],
    )


_GRP = 8
_CPA = _NWC // _NS
_NG = _CPA // _GRP


def _sc_agg_body(u_hbm, s_hbm, d_hbm, out_hbm, agg_sh, u_sh, zb, sidx_a,
                 sidx_b, didx_a, didx_b, rows_a, rows_b, gsem_a, gsem_b,
                 ssem_a, ssem_b):
    c = lax.axis_index("c")
    s = lax.axis_index("s")
    _fill_const(zb, _CH, _HW, 0.0)
    _zero_shared_slice(zb, agg_sh, s * _RPT)
    pltpu.sync_copy(u_hbm.at[c, pl.ds(s * _RPT, _RPT)],
                    u_sh.at[pl.ds(s * _RPT, _RPT)])
    plsc.subcore_barrier()
    base = s * _CPA
    sidx = (sidx_a, sidx_b)
    didx = (didx_a, didx_b)
    rows = (rows_a, rows_b)
    gsem = (gsem_a, gsem_b)
    ssem = (ssem_a, ssem_b)

    def load_idx(p, g):
        pltpu.sync_copy(s_hbm.at[pl.ds(base + g * _GRP, _GRP)], sidx[p])
        pltpu.sync_copy(d_hbm.at[pl.ds(base + g * _GRP, _GRP)], didx[p])

    def fire_gathers(p):
        for q in range(_GRP):
            pltpu.async_copy(u_sh.at[sidx[p].at[q]],
                             rows[p].at[pl.ds(q * _CH, _CH)], gsem[p])

    def wait_gathers(p):
        for q in range(_GRP):
            pltpu.make_async_copy(u_sh.at[sidx[p].at[q]],
                                  rows[p].at[pl.ds(q * _CH, _CH)],
                                  gsem[p]).wait()

    def fire_scatters(p):
        for q in range(_GRP):
            pltpu.async_copy(rows[p].at[pl.ds(q * _CH, _CH)],
                             agg_sh.at[didx[p].at[q]], ssem[p], add=True)

    def wait_scatters(p):
        for q in range(_GRP):
            pltpu.make_async_copy(rows[p].at[pl.ds(q * _CH, _CH)],
                                  agg_sh.at[didx[p].at[q]], ssem[p]).wait()

    load_idx(0, 0)
    fire_gathers(0)

    def step2(k, _):
        for b in (0, 1):
            g = 2 * k + b
            p, pn = b, 1 - b

            @pl.when(g > 0)
            def _():
                wait_scatters(pn)

            load_idx(pn, g + 1)
            fire_gathers(pn)
            wait_gathers(p)
            fire_scatters(p)
        return 0

    lax.fori_loop(0, _NG // 2, step2, 0)
    wait_scatters(1)
    wait_gathers(0)
    plsc.subcore_barrier()
    pltpu.sync_copy(agg_sh.at[pl.ds(s * _RPT, _RPT)],
                    out_hbm.at[c, pl.ds(s * _RPT, _RPT)])


@functools.cache
def _sc_agg():
    return pl.kernel(
        _sc_agg_body,
        out_type=jax.ShapeDtypeStruct((_NC, _NPAD, _HW), jnp.float32),
        mesh=_mesh(),
        compiler_params=pltpu.CompilerParams(use_tc_tiling_on_sc=False),
        scratch_types=[
            pltpu.VMEM_SHARED((_NPAD, _HW), jnp.float32),
            pltpu.VMEM_SHARED((_NPAD, _HW), jnp.float32),
            pltpu.VMEM((_CH, _HW), jnp.float32),
            pltpu.VMEM((_GRP, _CH), jnp.int32),
            pltpu.VMEM((_GRP, _CH), jnp.int32),
            pltpu.VMEM((_GRP, _CH), jnp.int32),
            pltpu.VMEM((_GRP, _CH), jnp.int32),
            pltpu.VMEM((_GRP * _CH, _HW), jnp.float32),
            pltpu.VMEM((_GRP * _CH, _HW), jnp.float32),
            pltpu.SemaphoreType.DMA,
            pltpu.SemaphoreType.DMA,
            pltpu.SemaphoreType.DMA,
            pltpu.SemaphoreType.DMA,
        ],
    )


def _tc1_body(x_ref, degp_ref, w1_ref, u1_ref, dinv_ref):
    deg = degp_ref[0, :, 0:1] + degp_ref[1, :, 0:1] + 1.0
    dinv = lax.rsqrt(deg)
    h = jnp.dot(x_ref[...], w1_ref[...], preferred_element_type=jnp.float32)
    u = h * dinv
    u1_ref[0] = u[:, :_HW]
    u1_ref[1] = u[:, _HW:]
    dinv_ref[...] = dinv


_tc1 = pl.pallas_call(
    _tc1_body,
    grid=(_NB,),
    in_specs=[
        pl.BlockSpec((_R, _D), lambda i: (i, 0)),
        pl.BlockSpec((_NC, _R, 8), lambda i: (0, i, 0)),
        pl.BlockSpec((_D, _H), lambda i: (0, 0)),
    ],
    out_specs=[
        pl.BlockSpec((_NC, _R, _HW), lambda i: (0, i, 0)),
        pl.BlockSpec((_R, 1), lambda i: (i, 0)),
    ],
    out_shape=[
        jax.ShapeDtypeStruct((_NC, _NPAD, _HW), jnp.float32),
        jax.ShapeDtypeStruct((_N, 1), jnp.float32),
    ],
)


def _tc2_body(aggp_ref, u1_ref, dinv_ref, b1_ref, w2_ref, u2_ref):
    aggu = jnp.concatenate([aggp_ref[0] + u1_ref[0], aggp_ref[1] + u1_ref[1]],
                           axis=1)
    t = aggu * dinv_ref[...] + b1_ref[...]
    t = jnp.maximum(t, 0.0)
    h2 = jnp.dot(t, w2_ref[...], preferred_element_type=jnp.float32)
    u = h2 * dinv_ref[...]
    u2_ref[0] = u[:, :_HW]
    u2_ref[1] = u[:, _HW:]


_tc2 = pl.pallas_call(
    _tc2_body,
    grid=(_NB,),
    in_specs=[
        pl.BlockSpec((_NC, _R, _HW), lambda i: (0, i, 0)),
        pl.BlockSpec((_NC, _R, _HW), lambda i: (0, i, 0)),
        pl.BlockSpec((_R, 1), lambda i: (i, 0)),
        pl.BlockSpec((1, _H), lambda i: (0, 0)),
        pl.BlockSpec((_H, _H), lambda i: (0, 0)),
    ],
    out_specs=pl.BlockSpec((_NC, _R, _HW), lambda i: (0, i, 0)),
    out_shape=jax.ShapeDtypeStruct((_NC, _NPAD, _HW), jnp.float32),
)


def _tc3_body(aggp_ref, u2_ref, dinv_ref, b2_ref, batch_ref, l1w_ref, l1b_ref,
              l2w_ref, l2b_ref, l3w_ref, l3b_ref, out_ref, sums_ref, cnts_ref):
    i = pl.program_id(0)
    aggu = jnp.concatenate([aggp_ref[0] + u2_ref[0], aggp_ref[1] + u2_ref[1]],
                           axis=1)
    out2 = aggu * dinv_ref[...] + b2_ref[...]
    out2 = jnp.maximum(out2, 0.0)
    b = batch_ref[0, 0, :]
    gids = lax.broadcasted_iota(jnp.int32, (_G, _R), 0)
    onehot = (b[None, :] == gids).astype(jnp.float32)
    psums = jnp.dot(onehot, out2, preferred_element_type=jnp.float32)
    pcnts = jnp.sum(onehot, axis=1, keepdims=True)

    @pl.when(i == 0)
    def _():
        sums_ref[...] = psums
        cnts_ref[...] = pcnts

    @pl.when(i > 0)
    def _():
        sums_ref[...] += psums
        cnts_ref[...] += pcnts

    @pl.when(i == _NB - 1)
    def _():
        g = sums_ref[...] / jnp.maximum(cnts_ref[...], 1.0)
        g = jnp.maximum(
            jnp.dot(g, l1w_ref[...], preferred_element_type=jnp.float32)
            + l1b_ref[...], 0.0)
        g = jnp.maximum(
            jnp.dot(g, l2w_ref[...], preferred_element_type=jnp.float32)
            + l2b_ref[...], 0.0)
        logits = (jnp.dot(g, l3w_ref[...], preferred_element_type=jnp.float32)
                  + l3b_ref[...])
        m = jnp.max(logits, axis=-1, keepdims=True)
        lse = m + jnp.log(jnp.sum(jnp.exp(logits - m), axis=-1, keepdims=True))
        out_ref[...] = logits - lse


_tc3 = pl.pallas_call(
    _tc3_body,
    grid=(_NB,),
    in_specs=[
        pl.BlockSpec((_NC, _R, _HW), lambda i: (0, i, 0)),
        pl.BlockSpec((_NC, _R, _HW), lambda i: (0, i, 0)),
        pl.BlockSpec((_R, 1), lambda i: (i, 0)),
        pl.BlockSpec((1, _H), lambda i: (0, 0)),
        pl.BlockSpec((1, 1, _R), lambda i: (i, 0, 0)),
        pl.BlockSpec((_H, _H), lambda i: (0, 0)),
        pl.BlockSpec((1, _H), lambda i: (0, 0)),
        pl.BlockSpec((_H, _H // 2), lambda i: (0, 0)),
        pl.BlockSpec((1, _H // 2), lambda i: (0, 0)),
        pl.BlockSpec((_H // 2, _C), lambda i: (0, 0)),
        pl.BlockSpec((1, _C), lambda i: (0, 0)),
    ],
    out_specs=pl.BlockSpec((_G, _C), lambda i: (0, 0)),
    out_shape=jax.ShapeDtypeStruct((_G, _C), jnp.float32),
    scratch_shapes=[
        pltpu.VMEM((_G, _H), jnp.float32),
        pltpu.VMEM((_G, 1), jnp.float32),
    ],
)


def kernel(x, edge_index, batch, W1, b1, W2, b2, lin1_W, lin1_b, lin2_W,
           lin2_b, lin3_W, lin3_b):
    src = edge_index[0]
    dst = edge_index[1]
    pad = _EPAD - _E + _DG * _CH
    scrap = _N + (jnp.arange(pad, dtype=jnp.int32) % (_NPAD - _N))
    srcp = jnp.concatenate([src, jnp.zeros((pad,), jnp.int32)]).reshape(-1, _CH)
    dstp = jnp.concatenate([dst, scrap]).reshape(-1, _CH)

    degp = _sc_deg()(dstp)
    u1, dinv = _tc1(x, degp, W1)
    agg1 = _sc_agg()(u1, srcp, dstp)
    u2 = _tc2(agg1, u1, dinv, b1.reshape(1, _H), W2)
    agg2 = _sc_agg()(u2, srcp, dstp)
    out = _tc3(agg2, u2, dinv, b2.reshape(1, _H), batch.reshape(_NB, 1, _R),
               lin1_W, lin1_b.reshape(1, _H), lin2_W,
               lin2_b.reshape(1, _H // 2), lin3_W, lin3_b.reshape(1, _C))
    return out
